# baseline (speedup 1.0000x reference)
import functools

import jax
import jax.numpy as jnp
from jax import lax
from jax.experimental import pallas as pl
from jax.experimental.pallas import tpu as pltpu
from jax.experimental.pallas import tpu_sc as plsc

Z_, N_, K_, DE, DV = 2, 2048, 32, 128, 128
ZN = Z_ * N_
B_ = ZN * K_
TROWS = 2 * ZN

BN = 128
BE = BN * K_


def _pack_bf16_pair(x):
    def bits(v):
        b = lax.bitcast_convert_type(v, jnp.int32)
        return (b + 0x7FFF + ((b >> 16) & 1)) >> 16
    return (bits(x[:, 0:64]) & 0xFFFF) | (bits(x[:, 64:128]) << 16)


def _gelu(x):
    return x * 0.5 * (1.0 + lax.erf(x * 0.7071067811865475))


def _ln(x, g, b):
    m = jnp.mean(x, axis=-1, keepdims=True)
    xc = x - m
    v = jnp.mean(xc * xc, axis=-1, keepdims=True)
    return xc * jax.lax.rsqrt(v + 1e-5) * g + b


def _tables_body(vn_ref, vo_ref, s_ref, k_ref, ar_ref, w0_ref,
                 t_ref, ai_ref, gidx_ref):
    w_vi = w0_ref[0:DV, :]
    w_vj = w0_ref[DV:2 * DV, :]
    w_es = w0_ref[3 * DV:4 * DV, :]
    for z in range(Z_):
        vn = vn_ref[z]
        vo = vo_ref[z]
        s = s_ref[z]
        t_ref[pl.ds(z * N_, N_), :] = _pack_bf16_pair(jnp.dot(
            vo, w_vj, preferred_element_type=jnp.float32))
        t_ref[pl.ds(ZN + z * N_, N_), :] = _pack_bf16_pair(
            jnp.dot(vn, w_vj, preferred_element_type=jnp.float32)
            + jnp.dot(s, w_es, preferred_element_type=jnp.float32))
        ai_ref[pl.ds(z * N_, N_), :] = jnp.dot(
            vn, w_vi, preferred_element_type=jnp.float32)
        gidx_ref[z] = k_ref[z] + z * N_ + ar_ref[z] * ZN


def _stage_a(V_new, V_old, S, Ki, ari, W0):
    return pl.pallas_call(
        _tables_body,
        out_shape=[
            jax.ShapeDtypeStruct((TROWS, DV // 2), jnp.int32),
            jax.ShapeDtypeStruct((ZN, DV), jnp.float32),
            jax.ShapeDtypeStruct((Z_, N_, K_), jnp.int32),
        ],
    )(V_new, V_old, S, Ki, ari, W0)


_CH = 128


def _sc_gather(table, gidx2d, nrows):
    info = plsc.get_sparse_core_info()
    nc, ns = info.num_cores, info.num_subcores
    nw = nc * ns
    rows_per_w = nrows // nw
    nch = rows_per_w // _CH
    mesh = plsc.VectorSubcoreMesh(core_axis_name="c", subcore_axis_name="s")

    @functools.partial(
        pl.kernel, mesh=mesh,
        out_type=jax.ShapeDtypeStruct((nrows, DV // 2), jnp.int32),
        scratch_types=[
            pltpu.VMEM((nch, _CH), jnp.int32),
            pltpu.VMEM((2, _CH, DV // 2), jnp.int32),
            pltpu.VMEM_SHARED((TROWS, DV // 2), jnp.int32),
            pltpu.SemaphoreType.DMA,
            pltpu.SemaphoreType.DMA,
        ],
    )
    def k(table_hbm, idx_hbm, out_hbm, idx_v, rows_v, table_sh, gsem, osem):
        sid = lax.axis_index("s")
        wid = sid * nc + lax.axis_index("c")
        base = wid * rows_per_w
        @pl.when(sid == 0)
        def _():
            pltpu.sync_copy(table_hbm, table_sh)
        plsc.subcore_barrier()
        pltpu.sync_copy(idx_hbm.at[pl.ds(wid * nch, nch)], idx_v)
        pltpu.async_copy(table_sh.at[idx_v.at[0]], rows_v.at[0], gsem)

        def body(c, carry):
            b = lax.rem(c, 2)
            pltpu.make_async_copy(
                table_sh.at[idx_v.at[c]], rows_v.at[b], gsem).wait()

            @pl.when(c + 1 < nch)
            def _():
                @pl.when(c >= 1)
                def _():
                    pltpu.make_async_copy(
                        rows_v.at[1 - b],
                        out_hbm.at[pl.ds(base + (c - 1) * _CH, _CH)],
                        osem).wait()
                pltpu.async_copy(
                    table_sh.at[idx_v.at[c + 1]], rows_v.at[1 - b], gsem)

            pltpu.async_copy(
                rows_v.at[b], out_hbm.at[pl.ds(base + c * _CH, _CH)], osem)
            return carry

        lax.fori_loop(0, nch, body, 0)
        pltpu.make_async_copy(
            rows_v.at[0], out_hbm.at[pl.ds(base, _CH)], osem).wait()
        pltpu.make_async_copy(
            rows_v.at[0], out_hbm.at[pl.ds(base, _CH)], osem).wait()

    return k(table, gidx2d)


def _main_body(e_ref, g_ref, ai_ref, vn_ref, em_ref, w0e_ref, b0_ref,
               w1_ref, b1_ref, w2_ref, b2_ref, f0_ref, fb0_ref, f1_ref,
               fb1_ref, g1_ref, be1_ref, g2_ref, be2_ref, out_ref):
    f32 = jnp.float32
    h = jnp.dot(e_ref[...], w0e_ref[...], preferred_element_type=f32)
    g32 = g_ref[...]
    h = h + jnp.concatenate(
        [lax.bitcast_convert_type(g32 << 16, f32),
         lax.bitcast_convert_type(g32 & jnp.int32(-65536), f32)], axis=1)
    ai = ai_ref[...]
    h = h + jnp.reshape(
        jnp.broadcast_to(ai[:, None, :], (BN, K_, DV)), (BE, DV))
    h = _gelu(h + b0_ref[0])
    h = _gelu(jnp.dot(h, w1_ref[...], preferred_element_type=f32) + b1_ref[0])
    m = jnp.dot(h, w2_ref[...], preferred_element_type=f32) + b2_ref[0]
    em3 = jnp.broadcast_to(em_ref[...][:, :, None], (BN, K_, DV))
    msum = jnp.sum(jnp.reshape(m, (BN, K_, DV)) * em3, axis=1)
    v1 = _ln(vn_ref[...] + msum, g1_ref[0], be1_ref[0])
    ff = jnp.dot(
        _gelu(jnp.dot(v1, f0_ref[...], preferred_element_type=f32)
              + fb0_ref[0]),
        f1_ref[...], preferred_element_type=f32) + fb1_ref[0]
    out_ref[...] = _ln(v1 + ff, g2_ref[0], be2_ref[0])


def _stage_c(E_flat, G, Ai, vn_flat, em_flat, w0e, b0, W1, b1, W2, b2,
             F0, fb0, F1, fb1, g1, be1, g2, be2):
    nn = Ai.shape[0]
    full = lambda shape: pl.BlockSpec(shape, lambda i: (0, 0))
    return pl.pallas_call(
        _main_body,
        grid=(nn // BN,),
        in_specs=[
            pl.BlockSpec((BE, DE), lambda i: (i, 0)),
            pl.BlockSpec((BE, DV // 2), lambda i: (i, 0)),
            pl.BlockSpec((BN, DV), lambda i: (i, 0)),
            pl.BlockSpec((BN, DV), lambda i: (i, 0)),
            pl.BlockSpec((BN, K_), lambda i: (i, 0)),
            full((DV, DV)), full((1, DV)),
            full((DV, DV)), full((1, DV)),
            full((DV, DV)), full((1, DV)),
            full((DV, 4 * DV)), full((1, 4 * DV)),
            full((4 * DV, DV)), full((1, DV)),
            full((1, DV)), full((1, DV)),
            full((1, DV)), full((1, DV)),
        ],
        out_specs=pl.BlockSpec((BN, DV), lambda i: (i, 0)),
        out_shape=jax.ShapeDtypeStruct((nn, DV), jnp.float32),
    )(E_flat, G, Ai, vn_flat, em_flat, w0e, b0, W1, b1, W2, b2,
      F0, fb0, F1, fb1, g1, be1, g2, be2)


def kernel(V_new, V_old, E, K, S, edge_mask, autoregressive_mask,
           W0, b0, W1, b1, W2, b2, F0, fb0, F1, fb1, g1, be1, g2, be2):
    Ki = K.astype(jnp.int32)
    ari = autoregressive_mask.astype(jnp.int32)

    T, Ai, gidx = _stage_a(V_new, V_old, S, Ki, ari, W0)

    G = _sc_gather(T, gidx.reshape(B_ // _CH, _CH), B_)

    out = _stage_c(
        E.reshape(B_, DE), G, Ai, V_new.reshape(ZN, DV),
        edge_mask.reshape(ZN, K_), W0[2 * DV:3 * DV],
        b0.reshape(1, DV), W1, b1.reshape(1, DV), W2, b2.reshape(1, DV),
        F0, fb0.reshape(1, 4 * DV), F1, fb1.reshape(1, DV),
        g1.reshape(1, DV), be1.reshape(1, DV),
        g2.reshape(1, DV), be2.reshape(1, DV))
    return out.reshape(Z_, N_, DV)


# speedup vs baseline: 1.0151x; 1.0151x over previous
import functools

import jax
import jax.numpy as jnp
from jax import lax
from jax.experimental import pallas as pl
from jax.experimental.pallas import tpu as pltpu
from jax.experimental.pallas import tpu_sc as plsc

Z_, N_, K_, DE, DV = 2, 2048, 32, 128, 128
ZN = Z_ * N_
B_ = ZN * K_
TROWS = 2 * ZN

BN = 128
BE = BN * K_


def _gelu(x):
    return x * 0.5 * (1.0 + lax.erf(x * 0.7071067811865475))


def _ln(x, g, b):
    m = jnp.mean(x, axis=-1, keepdims=True)
    xc = x - m
    v = jnp.mean(xc * xc, axis=-1, keepdims=True)
    return xc * jax.lax.rsqrt(v + 1e-5) * g + b


def _tables_body(vn_ref, vo_ref, s_ref, k_ref, ar_ref, w0_ref,
                 t_ref, ai_ref, gidx_ref):
    w_vi = w0_ref[0:DV, :]
    w_vj = w0_ref[DV:2 * DV, :]
    w_es = w0_ref[3 * DV:4 * DV, :]
    for z in range(Z_):
        vn = vn_ref[z]
        vo = vo_ref[z]
        s = s_ref[z]
        t_ref[pl.ds(z * N_, N_), :] = jnp.dot(
            vo, w_vj, preferred_element_type=jnp.float32)
        t_ref[pl.ds(ZN + z * N_, N_), :] = (
            jnp.dot(vn, w_vj, preferred_element_type=jnp.float32)
            + jnp.dot(s, w_es, preferred_element_type=jnp.float32))
        ai_ref[pl.ds(z * N_, N_), :] = jnp.dot(
            vn, w_vi, preferred_element_type=jnp.float32)
        gidx_ref[z] = k_ref[z] + z * N_ + ar_ref[z] * ZN


def _stage_a(V_new, V_old, S, Ki, ari, W0):
    return pl.pallas_call(
        _tables_body,
        out_shape=[
            jax.ShapeDtypeStruct((TROWS, DV), jnp.float32),
            jax.ShapeDtypeStruct((ZN, DV), jnp.float32),
            jax.ShapeDtypeStruct((Z_, N_, K_), jnp.int32),
        ],
    )(V_new, V_old, S, Ki, ari, W0)


_CH = 128


def _sc_gather(table, gidx2d, nrows):
    info = plsc.get_sparse_core_info()
    nc, ns = info.num_cores, info.num_subcores
    nw = nc * ns
    rows_per_w = nrows // nw
    nch = rows_per_w // _CH
    mesh = plsc.VectorSubcoreMesh(core_axis_name="c", subcore_axis_name="s")

    @functools.partial(
        pl.kernel, mesh=mesh,
        out_type=jax.ShapeDtypeStruct((nrows, DV), jnp.float32),
        scratch_types=[
            pltpu.VMEM((nch, _CH), jnp.int32),
            pltpu.VMEM((2, _CH, DV), jnp.float32),
            pltpu.VMEM_SHARED((TROWS, DV), jnp.float32),
            pltpu.SemaphoreType.DMA,
            pltpu.SemaphoreType.DMA,
        ],
    )
    def k(table_hbm, idx_hbm, out_hbm, idx_v, rows_v, table_sh, gsem, osem):
        sid = lax.axis_index("s")
        wid = sid * nc + lax.axis_index("c")
        base = wid * rows_per_w
        @pl.when(sid == 0)
        def _():
            pltpu.sync_copy(table_hbm, table_sh)
        plsc.subcore_barrier()
        pltpu.sync_copy(idx_hbm.at[pl.ds(wid * nch, nch)], idx_v)
        pltpu.async_copy(table_sh.at[idx_v.at[0]], rows_v.at[0], gsem)

        def body(c, carry):
            b = lax.rem(c, 2)
            pltpu.make_async_copy(
                table_sh.at[idx_v.at[c]], rows_v.at[b], gsem).wait()

            @pl.when(c + 1 < nch)
            def _():
                @pl.when(c >= 1)
                def _():
                    pltpu.make_async_copy(
                        rows_v.at[1 - b],
                        out_hbm.at[pl.ds(base + (c - 1) * _CH, _CH)],
                        osem).wait()
                pltpu.async_copy(
                    table_sh.at[idx_v.at[c + 1]], rows_v.at[1 - b], gsem)

            pltpu.async_copy(
                rows_v.at[b], out_hbm.at[pl.ds(base + c * _CH, _CH)], osem)
            return carry

        lax.fori_loop(0, nch, body, 0)
        pltpu.make_async_copy(
            rows_v.at[0], out_hbm.at[pl.ds(base, _CH)], osem).wait()
        pltpu.make_async_copy(
            rows_v.at[0], out_hbm.at[pl.ds(base, _CH)], osem).wait()

    return k(table, gidx2d)


def _main_body(e_ref, g_ref, ai_ref, vn_ref, em_ref, w0e_ref, b0_ref,
               w1_ref, b1_ref, w2_ref, b2_ref, f0_ref, fb0_ref, f1_ref,
               fb1_ref, g1_ref, be1_ref, g2_ref, be2_ref, out_ref):
    f32 = jnp.float32
    h = jnp.dot(e_ref[...], w0e_ref[...], preferred_element_type=f32)
    h = h + g_ref[...]
    ai = ai_ref[...]
    h = h + jnp.reshape(
        jnp.broadcast_to(ai[:, None, :], (BN, K_, DV)), (BE, DV))
    h = _gelu(h + b0_ref[0])
    h = _gelu(jnp.dot(h, w1_ref[...], preferred_element_type=f32) + b1_ref[0])
    m = jnp.dot(h, w2_ref[...], preferred_element_type=f32) + b2_ref[0]
    em3 = jnp.broadcast_to(em_ref[...][:, :, None], (BN, K_, DV))
    msum = jnp.sum(jnp.reshape(m, (BN, K_, DV)) * em3, axis=1)
    v1 = _ln(vn_ref[...] + msum, g1_ref[0], be1_ref[0])
    ff = jnp.dot(
        _gelu(jnp.dot(v1, f0_ref[...], preferred_element_type=f32)
              + fb0_ref[0]),
        f1_ref[...], preferred_element_type=f32) + fb1_ref[0]
    out_ref[...] = _ln(v1 + ff, g2_ref[0], be2_ref[0])


def _stage_c(E_flat, G, Ai, vn_flat, em_flat, w0e, b0, W1, b1, W2, b2,
             F0, fb0, F1, fb1, g1, be1, g2, be2):
    nn = Ai.shape[0]
    full = lambda shape: pl.BlockSpec(shape, lambda i: (0, 0))
    return pl.pallas_call(
        _main_body,
        grid=(nn // BN,),
        in_specs=[
            pl.BlockSpec((BE, DE), lambda i: (i, 0)),
            pl.BlockSpec((BE, DV), lambda i: (i, 0)),
            pl.BlockSpec((BN, DV), lambda i: (i, 0)),
            pl.BlockSpec((BN, DV), lambda i: (i, 0)),
            pl.BlockSpec((BN, K_), lambda i: (i, 0)),
            full((DV, DV)), full((1, DV)),
            full((DV, DV)), full((1, DV)),
            full((DV, DV)), full((1, DV)),
            full((DV, 4 * DV)), full((1, 4 * DV)),
            full((4 * DV, DV)), full((1, DV)),
            full((1, DV)), full((1, DV)),
            full((1, DV)), full((1, DV)),
        ],
        out_specs=pl.BlockSpec((BN, DV), lambda i: (i, 0)),
        out_shape=jax.ShapeDtypeStruct((nn, DV), jnp.float32),
    )(E_flat, G, Ai, vn_flat, em_flat, w0e, b0, W1, b1, W2, b2,
      F0, fb0, F1, fb1, g1, be1, g2, be2)


def kernel(V_new, V_old, E, K, S, edge_mask, autoregressive_mask,
           W0, b0, W1, b1, W2, b2, F0, fb0, F1, fb1, g1, be1, g2, be2):
    Ki = K.astype(jnp.int32)
    ari = autoregressive_mask.astype(jnp.int32)

    T, Ai, gidx = _stage_a(V_new, V_old, S, Ki, ari, W0)

    G = _sc_gather(T, gidx.reshape(B_ // _CH, _CH), B_)

    out = _stage_c(
        E.reshape(B_, DE), G, Ai, V_new.reshape(ZN, DV),
        edge_mask.reshape(ZN, K_), W0[2 * DV:3 * DV],
        b0.reshape(1, DV), W1, b1.reshape(1, DV), W2, b2.reshape(1, DV),
        F0, fb0.reshape(1, 4 * DV), F1, fb1.reshape(1, DV),
        g1.reshape(1, DV), be1.reshape(1, DV),
        g2.reshape(1, DV), be2.reshape(1, DV))
    return out.reshape(Z_, N_, DV)


# speedup vs baseline: 1.0909x; 1.0748x over previous
import functools

import jax
import jax.numpy as jnp
from jax import lax
from jax.experimental import pallas as pl
from jax.experimental.pallas import tpu as pltpu
from jax.experimental.pallas import tpu_sc as plsc

Z_, N_, K_, DE, DV = 2, 2048, 32, 128, 128
ZN = Z_ * N_
B_ = ZN * K_
TROWS = 2 * ZN

BN = 256
BE = BN * K_


def _gelu(x):
    return x * 0.5 * (1.0 + lax.erf(x * 0.7071067811865475))


def _ln(x, g, b):
    m = jnp.mean(x, axis=-1, keepdims=True)
    xc = x - m
    v = jnp.mean(xc * xc, axis=-1, keepdims=True)
    return xc * jax.lax.rsqrt(v + 1e-5) * g + b


def _tables_body(vn_ref, vo_ref, s_ref, k_ref, ar_ref, w0_ref,
                 t_ref, ai_ref, gidx_ref):
    w_vi = w0_ref[0:DV, :]
    w_vj = w0_ref[DV:2 * DV, :]
    w_es = w0_ref[3 * DV:4 * DV, :]
    for z in range(Z_):
        vn = vn_ref[z]
        vo = vo_ref[z]
        s = s_ref[z]
        t_ref[pl.ds(z * N_, N_), :] = jnp.dot(
            vo, w_vj, preferred_element_type=jnp.float32)
        t_ref[pl.ds(ZN + z * N_, N_), :] = (
            jnp.dot(vn, w_vj, preferred_element_type=jnp.float32)
            + jnp.dot(s, w_es, preferred_element_type=jnp.float32))
        ai_ref[pl.ds(z * N_, N_), :] = jnp.dot(
            vn, w_vi, preferred_element_type=jnp.float32)
        gidx_ref[z] = k_ref[z] + z * N_ + ar_ref[z] * ZN


def _stage_a(V_new, V_old, S, Ki, ari, W0):
    return pl.pallas_call(
        _tables_body,
        out_shape=[
            jax.ShapeDtypeStruct((TROWS, DV), jnp.float32),
            jax.ShapeDtypeStruct((ZN, DV), jnp.float32),
            jax.ShapeDtypeStruct((Z_, N_, K_), jnp.int32),
        ],
    )(V_new, V_old, S, Ki, ari, W0)


_CH = 128


def _sc_gather(table, gidx2d, nrows):
    info = plsc.get_sparse_core_info()
    nc, ns = info.num_cores, info.num_subcores
    nw = nc * ns
    rows_per_w = nrows // nw
    nch = rows_per_w // _CH
    mesh = plsc.VectorSubcoreMesh(core_axis_name="c", subcore_axis_name="s")

    @functools.partial(
        pl.kernel, mesh=mesh,
        out_type=jax.ShapeDtypeStruct((nrows, DV), jnp.float32),
        scratch_types=[
            pltpu.VMEM((nch, _CH), jnp.int32),
            pltpu.VMEM((2, _CH, DV), jnp.float32),
            pltpu.VMEM_SHARED((TROWS, DV), jnp.float32),
            pltpu.SemaphoreType.DMA,
            pltpu.SemaphoreType.DMA,
        ],
    )
    def k(table_hbm, idx_hbm, out_hbm, idx_v, rows_v, table_sh, gsem, osem):
        sid = lax.axis_index("s")
        wid = sid * nc + lax.axis_index("c")
        base = wid * rows_per_w
        @pl.when(sid == 0)
        def _():
            pltpu.sync_copy(table_hbm, table_sh)
        plsc.subcore_barrier()
        pltpu.sync_copy(idx_hbm.at[pl.ds(wid * nch, nch)], idx_v)
        pltpu.async_copy(table_sh.at[idx_v.at[0]], rows_v.at[0], gsem)

        def body(c, carry):
            b = lax.rem(c, 2)
            pltpu.make_async_copy(
                table_sh.at[idx_v.at[c]], rows_v.at[b], gsem).wait()

            @pl.when(c + 1 < nch)
            def _():
                @pl.when(c >= 1)
                def _():
                    pltpu.make_async_copy(
                        rows_v.at[1 - b],
                        out_hbm.at[pl.ds(base + (c - 1) * _CH, _CH)],
                        osem).wait()
                pltpu.async_copy(
                    table_sh.at[idx_v.at[c + 1]], rows_v.at[1 - b], gsem)

            pltpu.async_copy(
                rows_v.at[b], out_hbm.at[pl.ds(base + c * _CH, _CH)], osem)
            return carry

        lax.fori_loop(0, nch, body, 0)
        pltpu.make_async_copy(
            rows_v.at[0], out_hbm.at[pl.ds(base, _CH)], osem).wait()
        pltpu.make_async_copy(
            rows_v.at[0], out_hbm.at[pl.ds(base, _CH)], osem).wait()

    return k(table, gidx2d)


def _main_body(e_ref, g_ref, ai_ref, vn_ref, em_ref, w0e_ref, b0_ref,
               w1_ref, b1_ref, w2_ref, b2_ref, f0_ref, fb0_ref, f1_ref,
               fb1_ref, g1_ref, be1_ref, g2_ref, be2_ref, out_ref):
    f32 = jnp.float32
    h = jnp.dot(e_ref[...], w0e_ref[...], preferred_element_type=f32)
    h = h + g_ref[...]
    ai = ai_ref[...]
    h = h + jnp.reshape(
        jnp.broadcast_to(ai[:, None, :], (BN, K_, DV)), (BE, DV))
    h = _gelu(h + b0_ref[0])
    h = _gelu(jnp.dot(h, w1_ref[...], preferred_element_type=f32) + b1_ref[0])
    m = jnp.dot(h, w2_ref[...], preferred_element_type=f32) + b2_ref[0]
    em3 = jnp.broadcast_to(em_ref[...][:, :, None], (BN, K_, DV))
    msum = jnp.sum(jnp.reshape(m, (BN, K_, DV)) * em3, axis=1)
    v1 = _ln(vn_ref[...] + msum, g1_ref[0], be1_ref[0])
    ff = jnp.dot(
        _gelu(jnp.dot(v1, f0_ref[...], preferred_element_type=f32)
              + fb0_ref[0]),
        f1_ref[...], preferred_element_type=f32) + fb1_ref[0]
    out_ref[...] = _ln(v1 + ff, g2_ref[0], be2_ref[0])


def _stage_c(E_flat, G, Ai, vn_flat, em_flat, w0e, b0, W1, b1, W2, b2,
             F0, fb0, F1, fb1, g1, be1, g2, be2):
    nn = Ai.shape[0]
    full = lambda shape: pl.BlockSpec(shape, lambda i: (0, 0))
    return pl.pallas_call(
        _main_body,
        grid=(nn // BN,),
        in_specs=[
            pl.BlockSpec((BE, DE), lambda i: (i, 0)),
            pl.BlockSpec((BE, DV), lambda i: (i, 0)),
            pl.BlockSpec((BN, DV), lambda i: (i, 0)),
            pl.BlockSpec((BN, DV), lambda i: (i, 0)),
            pl.BlockSpec((BN, K_), lambda i: (i, 0)),
            full((DV, DV)), full((1, DV)),
            full((DV, DV)), full((1, DV)),
            full((DV, DV)), full((1, DV)),
            full((DV, 4 * DV)), full((1, 4 * DV)),
            full((4 * DV, DV)), full((1, DV)),
            full((1, DV)), full((1, DV)),
            full((1, DV)), full((1, DV)),
        ],
        out_specs=pl.BlockSpec((BN, DV), lambda i: (i, 0)),
        out_shape=jax.ShapeDtypeStruct((nn, DV), jnp.float32),
    )(E_flat, G, Ai, vn_flat, em_flat, w0e, b0, W1, b1, W2, b2,
      F0, fb0, F1, fb1, g1, be1, g2, be2)


def kernel(V_new, V_old, E, K, S, edge_mask, autoregressive_mask,
           W0, b0, W1, b1, W2, b2, F0, fb0, F1, fb1, g1, be1, g2, be2):
    Ki = K.astype(jnp.int32)
    ari = autoregressive_mask.astype(jnp.int32)

    T, Ai, gidx = _stage_a(V_new, V_old, S, Ki, ari, W0)

    G = _sc_gather(T, gidx.reshape(B_ // _CH, _CH), B_)

    out = _stage_c(
        E.reshape(B_, DE), G, Ai, V_new.reshape(ZN, DV),
        edge_mask.reshape(ZN, K_), W0[2 * DV:3 * DV],
        b0.reshape(1, DV), W1, b1.reshape(1, DV), W2, b2.reshape(1, DV),
        F0, fb0.reshape(1, 4 * DV), F1, fb1.reshape(1, DV),
        g1.reshape(1, DV), be1.reshape(1, DV),
        g2.reshape(1, DV), be2.reshape(1, DV))
    return out.reshape(Z_, N_, DV)
